# baseline (speedup 1.0000x reference)
import jax
import jax.numpy as jnp
from jax.experimental import pallas as pl
from jax.experimental.pallas import tpu as pltpu

REF_TOKEN_ID = 32000
SEG_OFF = 256
R_MAX = 16


def _index_body(ids_ref, idx_ref, cnt_ref):
    ids = ids_ref[...]
    B, S = ids.shape
    pos = jax.lax.broadcasted_iota(jnp.int32, (B, S), 1)
    mask = (ids == REF_TOKEN_ID) & (pos >= 1)
    mi = mask.astype(jnp.int32)
    cum = mi
    k = 1
    while k < S:
        shifted = jnp.concatenate(
            [jnp.zeros((B, k), jnp.int32), cum[:, : S - k]], axis=1)
        cum = cum + shifted
        k *= 2
    cnt_ref[...] = cum[:, S - 1:S]
    cols = []
    for r in range(R_MAX):
        sel = mask & (cum == (r + 1))
        cols.append(jnp.sum(jnp.where(sel, pos, 0), axis=1, keepdims=True))
    idx_ref[...] = jnp.concatenate(cols, axis=1) + SEG_OFF


def _gm_body(idx_ref, cnt_ref, hs_ref, w_hbm, b_ref, out_ref,
             scr, w_v, sem, w_sem):
    B = 8
    NWC = 16
    D = w_v.shape[0]
    wchunks = []
    dchunk = D // NWC
    for q in range(NWC):
        wc = pltpu.make_async_copy(
            w_hbm.at[pl.ds(q * dchunk, dchunk), :],
            w_v.at[pl.ds(q * dchunk, dchunk), :], w_sem)
        wc.start()
        wchunks.append(wc)
    def _gcopy(b, r):
        return pltpu.make_async_copy(
            hs_ref.at[pl.ds(idx_ref[b, r], 1), b, :],
            scr.at[pl.ds(b * R_MAX + r, 1), :], sem)
    for b in range(B):
        for r in range(R_MAX):
            @pl.when(r < cnt_ref[b, 0])
            def _(b=b, r=r):
                _gcopy(b, r).start()
    bias = b_ref[...]
    riota = jax.lax.broadcasted_iota(jnp.int32, (R_MAX, 1), 0)
    for b in range(B):
        for r in range(R_MAX):
            @pl.when(r < cnt_ref[b, 0])
            def _(b=b, r=r):
                _gcopy(b, r).wait()
    DG = out_ref.shape[2]
    acc = jnp.zeros((B * R_MAX, DG), jnp.float32)
    for q in range(NWC):
        wchunks[q].wait()
        acc = acc + jnp.dot(scr[:, pl.ds(q * dchunk, dchunk)],
                            w_v[pl.ds(q * dchunk, dchunk), :],
                            preferred_element_type=jnp.float32)
    for b in range(B):
        y = acc[b * R_MAX:(b + 1) * R_MAX, :] + bias
        out_ref[b] = jnp.where(riota < cnt_ref[b, 0], y, 0.0)


def kernel(input_ids, last_hidden_state, W_proj, b_proj):
    B, S = input_ids.shape
    _, L, D = last_hidden_state.shape
    DG = W_proj.shape[1]
    ids32 = input_ids.astype(jnp.int32)

    idx, cnt = pl.pallas_call(
        _index_body,
        out_shape=(
            jax.ShapeDtypeStruct((B, R_MAX), jnp.int32),
            jax.ShapeDtypeStruct((B, 1), jnp.int32),
        ),
    )(ids32)

    out = pl.pallas_call(
        _gm_body,
        in_specs=[
            pl.BlockSpec(memory_space=pltpu.SMEM),
            pl.BlockSpec(memory_space=pltpu.SMEM),
            pl.BlockSpec(memory_space=pl.ANY),
            pl.BlockSpec(memory_space=pl.ANY),
            pl.BlockSpec((1, DG), lambda: (0, 0)),
        ],
        out_specs=pl.BlockSpec((B, R_MAX, DG), lambda: (0, 0, 0)),
        out_shape=jax.ShapeDtypeStruct((B, R_MAX, DG), jnp.float32),
        scratch_shapes=[
            pltpu.VMEM((B * R_MAX, D), jnp.float32),
            pltpu.VMEM((D, DG), jnp.float32),
            pltpu.SemaphoreType.DMA,
            pltpu.SemaphoreType.DMA,
        ],
    )(idx, cnt, jnp.transpose(last_hidden_state, (1, 0, 2)), W_proj,
      b_proj.reshape(1, DG))
    return out


# speedup vs baseline: 1.0815x; 1.0815x over previous
import jax
import jax.numpy as jnp
from jax.experimental import pallas as pl
from jax.experimental.pallas import tpu as pltpu

REF_TOKEN_ID = 32000
SEG_OFF = 256
R_MAX = 16


def _index_body(ids_ref, idx_ref, cnt_ref):
    ids = ids_ref[...]
    B, S = ids.shape
    pos = jax.lax.broadcasted_iota(jnp.int32, (B, S), 1)
    mask = (ids == REF_TOKEN_ID) & (pos >= 1)
    mi = mask.astype(jnp.int32)
    cum = mi
    k = 1
    while k < S:
        shifted = jnp.concatenate(
            [jnp.zeros((B, k), jnp.int32), cum[:, : S - k]], axis=1)
        cum = cum + shifted
        k *= 2
    cnt_ref[...] = cum[:, S - 1:S]
    cols = []
    for r in range(R_MAX):
        sel = mask & (cum == (r + 1))
        cols.append(jnp.sum(jnp.where(sel, pos, 0), axis=1, keepdims=True))
    idx_ref[...] = jnp.concatenate(cols, axis=1) + SEG_OFF


def _gm_body(idx_ref, cnt_ref, hs_ref, w_hbm, b_ref, out_ref,
             scr, w_v, sem, w_sem):
    B = 8
    NWC = 4
    D = w_v.shape[0]
    wchunks = []
    dchunk = D // NWC
    for q in range(NWC):
        wc = pltpu.make_async_copy(
            w_hbm.at[pl.ds(q * dchunk, dchunk), :],
            w_v.at[pl.ds(q * dchunk, dchunk), :], w_sem)
        wc.start()
        wchunks.append(wc)
    def _gcopy(b, r):
        return pltpu.make_async_copy(
            hs_ref.at[pl.ds(idx_ref[b, r], 1), b, :],
            scr.at[pl.ds(b * R_MAX + r, 1), :], sem)
    for b in range(B):
        for r in range(R_MAX):
            @pl.when(r < cnt_ref[b, 0])
            def _(b=b, r=r):
                _gcopy(b, r).start()
    bias = b_ref[...]
    riota = jax.lax.broadcasted_iota(jnp.int32, (R_MAX, 1), 0)
    for b in range(B):
        for r in range(R_MAX):
            @pl.when(r < cnt_ref[b, 0])
            def _(b=b, r=r):
                _gcopy(b, r).wait()
    DG = out_ref.shape[2]
    acc = jnp.zeros((B * R_MAX, DG), jnp.float32)
    for q in range(NWC):
        wchunks[q].wait()
        acc = acc + jnp.dot(scr[:, pl.ds(q * dchunk, dchunk)],
                            w_v[pl.ds(q * dchunk, dchunk), :],
                            preferred_element_type=jnp.float32)
    for b in range(B):
        y = acc[b * R_MAX:(b + 1) * R_MAX, :] + bias
        out_ref[b] = jnp.where(riota < cnt_ref[b, 0], y, 0.0)


def kernel(input_ids, last_hidden_state, W_proj, b_proj):
    B, S = input_ids.shape
    _, L, D = last_hidden_state.shape
    DG = W_proj.shape[1]
    ids32 = input_ids.astype(jnp.int32)

    idx, cnt = pl.pallas_call(
        _index_body,
        out_shape=(
            jax.ShapeDtypeStruct((B, R_MAX), jnp.int32),
            jax.ShapeDtypeStruct((B, 1), jnp.int32),
        ),
    )(ids32)

    out = pl.pallas_call(
        _gm_body,
        in_specs=[
            pl.BlockSpec(memory_space=pltpu.SMEM),
            pl.BlockSpec(memory_space=pltpu.SMEM),
            pl.BlockSpec(memory_space=pl.ANY),
            pl.BlockSpec(memory_space=pl.ANY),
            pl.BlockSpec((1, DG), lambda: (0, 0)),
        ],
        out_specs=pl.BlockSpec((B, R_MAX, DG), lambda: (0, 0, 0)),
        out_shape=jax.ShapeDtypeStruct((B, R_MAX, DG), jnp.float32),
        scratch_shapes=[
            pltpu.VMEM((B * R_MAX, D), jnp.float32),
            pltpu.VMEM((D, DG), jnp.float32),
            pltpu.SemaphoreType.DMA,
            pltpu.SemaphoreType.DMA,
        ],
    )(idx, cnt, jnp.transpose(last_hidden_state, (1, 0, 2)), W_proj,
      b_proj.reshape(1, DG))
    return out


# speedup vs baseline: 1.0882x; 1.0062x over previous
import jax
import jax.numpy as jnp
from jax.experimental import pallas as pl
from jax.experimental.pallas import tpu as pltpu

REF_TOKEN_ID = 32000
SEG_OFF = 256
R_MAX = 16


def _index_body(ids_ref, idx_ref, cnt_ref):
    ids = ids_ref[...]
    B, S = ids.shape
    pos = jax.lax.broadcasted_iota(jnp.int32, (B, S), 1)
    mask = (ids == REF_TOKEN_ID) & (pos >= 1)
    mi = mask.astype(jnp.int32)
    cum = mi
    k = 1
    while k < S:
        shifted = jnp.concatenate(
            [jnp.zeros((B, k), jnp.int32), cum[:, : S - k]], axis=1)
        cum = cum + shifted
        k *= 2
    cnt_ref[...] = cum[:, S - 1:S]
    cols = []
    for r in range(R_MAX):
        sel = mask & (cum == (r + 1))
        cols.append(jnp.sum(jnp.where(sel, pos, 0), axis=1, keepdims=True))
    idx_ref[...] = jnp.concatenate(cols, axis=1) + SEG_OFF


def _gm_body(idx_ref, cnt_ref, hs_ref, w_hbm, b_ref, out_ref,
             scr, w_v, sem, w_sem):
    B = 8
    NWC = 8
    D = w_v.shape[0]
    wchunks = []
    dchunk = D // NWC
    for q in range(NWC):
        wc = pltpu.make_async_copy(
            w_hbm.at[pl.ds(q * dchunk, dchunk), :],
            w_v.at[pl.ds(q * dchunk, dchunk), :], w_sem)
        wc.start()
        wchunks.append(wc)
    def _gcopy(b, r):
        return pltpu.make_async_copy(
            hs_ref.at[pl.ds(idx_ref[b, r], 1), b, :],
            scr.at[pl.ds(b * R_MAX + r, 1), :], sem)
    for b in range(B):
        for r in range(R_MAX):
            @pl.when(r < cnt_ref[b, 0])
            def _(b=b, r=r):
                _gcopy(b, r).start()
    bias = b_ref[...]
    riota = jax.lax.broadcasted_iota(jnp.int32, (R_MAX, 1), 0)
    for b in range(B):
        for r in range(R_MAX):
            @pl.when(r < cnt_ref[b, 0])
            def _(b=b, r=r):
                _gcopy(b, r).wait()
    DG = out_ref.shape[2]
    acc = jnp.zeros((B * R_MAX, DG), jnp.float32)
    for q in range(NWC):
        wchunks[q].wait()
        acc = acc + jnp.dot(scr[:, pl.ds(q * dchunk, dchunk)],
                            w_v[pl.ds(q * dchunk, dchunk), :],
                            preferred_element_type=jnp.float32)
    for b in range(B):
        y = acc[b * R_MAX:(b + 1) * R_MAX, :] + bias
        out_ref[b] = jnp.where(riota < cnt_ref[b, 0], y, 0.0)


def kernel(input_ids, last_hidden_state, W_proj, b_proj):
    B, S = input_ids.shape
    _, L, D = last_hidden_state.shape
    DG = W_proj.shape[1]
    ids32 = input_ids.astype(jnp.int32)

    idx, cnt = pl.pallas_call(
        _index_body,
        out_shape=(
            jax.ShapeDtypeStruct((B, R_MAX), jnp.int32),
            jax.ShapeDtypeStruct((B, 1), jnp.int32),
        ),
    )(ids32)

    out = pl.pallas_call(
        _gm_body,
        in_specs=[
            pl.BlockSpec(memory_space=pltpu.SMEM),
            pl.BlockSpec(memory_space=pltpu.SMEM),
            pl.BlockSpec(memory_space=pl.ANY),
            pl.BlockSpec(memory_space=pl.ANY),
            pl.BlockSpec((1, DG), lambda: (0, 0)),
        ],
        out_specs=pl.BlockSpec((B, R_MAX, DG), lambda: (0, 0, 0)),
        out_shape=jax.ShapeDtypeStruct((B, R_MAX, DG), jnp.float32),
        scratch_shapes=[
            pltpu.VMEM((B * R_MAX, D), jnp.float32),
            pltpu.VMEM((D, DG), jnp.float32),
            pltpu.SemaphoreType.DMA,
            pltpu.SemaphoreType.DMA,
        ],
    )(idx, cnt, jnp.transpose(last_hidden_state, (1, 0, 2)), W_proj,
      b_proj.reshape(1, DG))
    return out
